# baseline (speedup 1.0000x reference)
import functools
import jax
import jax.numpy as jnp
from jax import lax
from jax.experimental import pallas as pl
from jax.experimental.pallas import tpu as pltpu
from jax.experimental.pallas import tpu_sc as plsc

N = 10000
E = 320000
D = 128
DE = 16
H = 256
EPS = 128.0

NC = 2
NT = 16
CH = 64
CPT0 = 108
CPT1 = 212
EP = CH * NT * (CPT0 + CPT1)
NSP = 10240
RPT = NSP // NT


def _sc_aggregate(src2d, dst2d, xp, embs):
  mesh = plsc.VectorSubcoreMesh(core_axis_name="c", subcore_axis_name="s")

  @functools.partial(
      pl.kernel,
      out_type=jax.ShapeDtypeStruct((NC, NSP, D), jnp.float32),
      mesh=mesh,
      scratch_types=[
          pltpu.VMEM_SHARED((NSP, D), jnp.float32),
          pltpu.VMEM((2, 1, CH), jnp.int32),
          pltpu.VMEM((2, 1, CH), jnp.int32),
          pltpu.VMEM((2, CH, D), jnp.float32),
          pltpu.VMEM((2, CH, D), jnp.float32),
          pltpu.SemaphoreType.DMA,
          pltpu.SemaphoreType.DMA,
          pltpu.SemaphoreType.DMA,
          pltpu.SemaphoreType.DMA,
      ],
  )
  def k(src_h, dst_h, xp_h, embs_h, out_h, acc_s, srcv, dstv, rows, emb,
        sem_i0, sem_i1, sem_d0, sem_d1):
    c = lax.axis_index("c")
    s = lax.axis_index("s")
    sem_i = (sem_i0, sem_i1)
    sem_d = (sem_d0, sem_d1)

    def zbody(i, _):
      for q in range(D // 16):
        rows[0, i, pl.ds(q * 16, 16)] = jnp.zeros((16,), jnp.float32)
      return 0
    lax.fori_loop(0, CH, zbody, 0)
    arow = s * RPT
    for t in range(RPT // CH):
      pltpu.sync_copy(rows.at[0, pl.ds(0, CH)],
                      acc_s.at[pl.ds(arow + t * CH, CH)])

    plsc.subcore_barrier()

    cpt = jnp.where(c == 0, CPT0, CPT1)
    tile0 = jnp.where(c == 0, s * CPT0, NT * CPT0 + s * CPT1)

    def start_idx(j, b):
      pltpu.async_copy(src_h.at[pl.ds(tile0 + j, 1)], srcv.at[b], sem_i[b])
      pltpu.async_copy(dst_h.at[pl.ds(tile0 + j, 1)], dstv.at[b], sem_i[b])

    def wait_idx(b):
      pltpu.make_async_copy(src_h.at[pl.ds(0, 1)], srcv.at[b],
                            sem_i[b]).wait()
      pltpu.make_async_copy(dst_h.at[pl.ds(0, 1)], dstv.at[b],
                            sem_i[b]).wait()

    def start_data(j, b):
      pltpu.async_copy(xp_h.at[srcv.at[b, 0]], rows.at[b], sem_d[b])
      pltpu.async_copy(embs_h.at[pl.ds((tile0 + j) * CH, CH)], emb.at[b],
                       sem_d[b])

    def wait_data(b):
      pltpu.make_async_copy(xp_h.at[pl.ds(0, CH)], rows.at[b],
                            sem_d[b]).wait()
      pltpu.make_async_copy(embs_h.at[pl.ds(0, CH)], emb.at[b],
                            sem_d[b]).wait()

    def compute_scatter(b):
      def cbody(i, _):
        for r in range(4):
          for q in range(D // 16):
            sl = pl.ds(q * 16, 16)
            rows[b, i * 4 + r, sl] = jnp.maximum(
                rows[b, i * 4 + r, sl] + emb[b, i * 4 + r, sl], 0.0)
        return 0
      lax.fori_loop(0, CH // 4, cbody, 0)
      pltpu.sync_copy(rows.at[b], acc_s.at[dstv.at[b, 0]], add=True)

    start_idx(0, 0)
    start_idx(1, 1)
    wait_idx(0)
    start_data(0, 0)

    def body(jj2, _):
      for b in range(2):
        j = jj2 * 2 + b
        wait_data(b)
        wait_idx(1 - b)
        start_data(j + 1, 1 - b)
        compute_scatter(b)
        start_idx(j + 2, b)
      return 0
    lax.fori_loop(0, (cpt - 2) // 2, body, 0)

    wait_data(0)
    wait_idx(1)
    start_data(cpt - 1, 1)
    compute_scatter(0)
    wait_data(1)
    compute_scatter(1)

    plsc.subcore_barrier()

    pltpu.sync_copy(acc_s.at[pl.ds(s * RPT, RPT)],
                    out_h.at[c, pl.ds(s * RPT, RPT)])

  return k(src2d, dst2d, xp, embs)


def _edge_embed(edge_attr, We, be):
  blk = 1600
  grid = (E // blk,)

  def body(ea_ref, we_ref, be_ref, out_ref):
    out_ref[...] = (
        jnp.dot(ea_ref[...], we_ref[...], preferred_element_type=jnp.float32)
        + be_ref[...])

  return pl.pallas_call(
      body,
      grid=grid,
      in_specs=[
          pl.BlockSpec((blk, DE), lambda i: (i, 0)),
          pl.BlockSpec((DE, D), lambda i: (0, 0)),
          pl.BlockSpec((1, D), lambda i: (0, 0)),
      ],
      out_specs=pl.BlockSpec((blk, D), lambda i: (i, 0)),
      out_shape=jax.ShapeDtypeStruct((EP, D), jnp.float32),
  )(edge_attr, We, be.reshape(1, D))


def _layernorm(a, g, b):
  mu = jnp.mean(a, axis=-1, keepdims=True)
  var = jnp.mean(jnp.square(a - mu), axis=-1, keepdims=True)
  return (a - mu) * lax.rsqrt(var + 1e-5) * g + b


def _mlp(partials, xp, W1, b1, g, bt, W2, b2):
  blk = 2000
  grid = (N // blk,)

  def body(p_ref, x_ref, w1, b1r, gr, btr, w2, b2r, out_ref):
    h = (1.0 + EPS) * x_ref[...] + p_ref[0] + p_ref[1]
    a = jnp.dot(h, w1[...], preferred_element_type=jnp.float32) + b1r[...]
    a = _layernorm(a, gr[...], btr[...])
    a = jnp.maximum(a, 0.0)
    out_ref[...] = (
        jnp.dot(a, w2[...], preferred_element_type=jnp.float32) + b2r[...])

  return pl.pallas_call(
      body,
      grid=grid,
      in_specs=[
          pl.BlockSpec((NC, blk, D), lambda i: (0, i, 0)),
          pl.BlockSpec((blk, D), lambda i: (i, 0)),
          pl.BlockSpec((D, H), lambda i: (0, 0)),
          pl.BlockSpec((1, H), lambda i: (0, 0)),
          pl.BlockSpec((1, H), lambda i: (0, 0)),
          pl.BlockSpec((1, H), lambda i: (0, 0)),
          pl.BlockSpec((H, D), lambda i: (0, 0)),
          pl.BlockSpec((1, D), lambda i: (0, 0)),
      ],
      out_specs=pl.BlockSpec((blk, D), lambda i: (i, 0)),
      out_shape=jax.ShapeDtypeStruct((N, D), jnp.float32),
  )(partials, xp, W1, b1.reshape(1, H), g.reshape(1, H), bt.reshape(1, H),
    W2, b2.reshape(1, D))


def _mlp_final(partials, y1, W1, b1, g, bt, W2, b2, Wout, bout):
  blk = 2000
  grid = (N // blk,)

  def body(p_ref, y1_ref, w1, b1r, gr, btr, w2, b2r, wo, bo, out_ref):
    y1b = y1_ref[...]
    h = (1.0 + EPS) * y1b + p_ref[0] + p_ref[1]
    a = jnp.dot(h, w1[...], preferred_element_type=jnp.float32) + b1r[...]
    a = _layernorm(a, gr[...], btr[...])
    a = jnp.maximum(a, 0.0)
    y2 = jnp.dot(a, w2[...], preferred_element_type=jnp.float32) + b2r[...]
    out_ref[...] = (
        jnp.dot(y1b, wo[:D], preferred_element_type=jnp.float32)
        + jnp.dot(y2, wo[D:], preferred_element_type=jnp.float32)
        + bo[...])

  return pl.pallas_call(
      body,
      grid=grid,
      in_specs=[
          pl.BlockSpec((NC, blk, D), lambda i: (0, i, 0)),
          pl.BlockSpec((blk, D), lambda i: (i, 0)),
          pl.BlockSpec((D, H), lambda i: (0, 0)),
          pl.BlockSpec((1, H), lambda i: (0, 0)),
          pl.BlockSpec((1, H), lambda i: (0, 0)),
          pl.BlockSpec((1, H), lambda i: (0, 0)),
          pl.BlockSpec((H, D), lambda i: (0, 0)),
          pl.BlockSpec((1, D), lambda i: (0, 0)),
          pl.BlockSpec((2 * D, D), lambda i: (0, 0)),
          pl.BlockSpec((1, D), lambda i: (0, 0)),
      ],
      out_specs=pl.BlockSpec((blk, D), lambda i: (i, 0)),
      out_shape=jax.ShapeDtypeStruct((N, D), jnp.float32),
  )(partials, y1, W1, b1.reshape(1, H), g.reshape(1, H), bt.reshape(1, H),
    W2, b2.reshape(1, D), Wout, bout.reshape(1, D))


def kernel(x, edge_index, edge_attr,
           We0, be0, W10, b10, g0, bt0, W20, b20,
           We1, be1, W11, b11, g1, bt1, W21, b21,
           Wout, bout):
  src = edge_index[0]
  dst = edge_index[1]
  pad = EP - E
  src_p = jnp.concatenate([src, jnp.zeros((pad,), jnp.int32)])
  sink = N + jnp.arange(pad, dtype=jnp.int32) % (NSP - N)
  dst_p = jnp.concatenate([dst, sink])
  src2d = src_p.reshape(EP // CH, CH)
  dst2d = dst_p.reshape(EP // CH, CH)
  embs0 = _edge_embed(edge_attr, We0, be0)
  embs1 = _edge_embed(edge_attr, We1, be1)

  part0 = _sc_aggregate(src2d, dst2d, x, embs0)
  y1 = _mlp(part0, x, W10, b10, g0, bt0, W20, b20)
  part1 = _sc_aggregate(src2d, dst2d, y1, embs1)
  return _mlp_final(part1, y1, W11, b11, g1, bt1, W21, b21, Wout, bout)


# speedup vs baseline: 1.1118x; 1.1118x over previous
import functools
import jax
import jax.numpy as jnp
from jax import lax
from jax.experimental import pallas as pl
from jax.experimental.pallas import tpu as pltpu
from jax.experimental.pallas import tpu_sc as plsc

N = 10000
E = 320000
D = 128
DE = 16
H = 256
EPS = 128.0

NC = 2
NT = 16
CH = 64
CPT0 = 212
CPT1 = 108
EP = CH * NT * (CPT0 + CPT1)
NSP = 10240
RPT = NSP // NT


def _sc_aggregate(src2d, dst2d, xp, embs):
  mesh = plsc.VectorSubcoreMesh(core_axis_name="c", subcore_axis_name="s")

  @functools.partial(
      pl.kernel,
      out_type=jax.ShapeDtypeStruct((NC, NSP, D), jnp.float32),
      mesh=mesh,
      scratch_types=[
          pltpu.VMEM_SHARED((NSP, D), jnp.float32),
          pltpu.VMEM((2, 1, CH), jnp.int32),
          pltpu.VMEM((2, 1, CH), jnp.int32),
          pltpu.VMEM((2, CH, D), jnp.float32),
          pltpu.VMEM((2, CH, D), jnp.float32),
          pltpu.SemaphoreType.DMA,
          pltpu.SemaphoreType.DMA,
          pltpu.SemaphoreType.DMA,
          pltpu.SemaphoreType.DMA,
      ],
  )
  def k(src_h, dst_h, xp_h, embs_h, out_h, acc_s, srcv, dstv, rows, emb,
        sem_i0, sem_i1, sem_d0, sem_d1):
    c = lax.axis_index("c")
    s = lax.axis_index("s")
    sem_i = (sem_i0, sem_i1)
    sem_d = (sem_d0, sem_d1)

    def zbody(i, _):
      for q in range(D // 16):
        rows[0, i, pl.ds(q * 16, 16)] = jnp.zeros((16,), jnp.float32)
      return 0
    lax.fori_loop(0, CH, zbody, 0)
    arow = s * RPT
    for t in range(RPT // CH):
      pltpu.sync_copy(rows.at[0, pl.ds(0, CH)],
                      acc_s.at[pl.ds(arow + t * CH, CH)])

    plsc.subcore_barrier()

    cpt = jnp.where(c == 0, CPT0, CPT1)
    tile0 = jnp.where(c == 0, s * CPT0, NT * CPT0 + s * CPT1)

    def start_idx(j, b):
      pltpu.async_copy(src_h.at[pl.ds(tile0 + j, 1)], srcv.at[b], sem_i[b])
      pltpu.async_copy(dst_h.at[pl.ds(tile0 + j, 1)], dstv.at[b], sem_i[b])

    def wait_idx(b):
      pltpu.make_async_copy(src_h.at[pl.ds(0, 1)], srcv.at[b],
                            sem_i[b]).wait()
      pltpu.make_async_copy(dst_h.at[pl.ds(0, 1)], dstv.at[b],
                            sem_i[b]).wait()

    def start_data(j, b):
      pltpu.async_copy(xp_h.at[srcv.at[b, 0]], rows.at[b], sem_d[b])
      pltpu.async_copy(embs_h.at[pl.ds((tile0 + j) * CH, CH)], emb.at[b],
                       sem_d[b])

    def wait_data(b):
      pltpu.make_async_copy(xp_h.at[pl.ds(0, CH)], rows.at[b],
                            sem_d[b]).wait()
      pltpu.make_async_copy(embs_h.at[pl.ds(0, CH)], emb.at[b],
                            sem_d[b]).wait()

    def compute_scatter(b):
      def cbody(i, _):
        for r in range(4):
          for q in range(D // 16):
            sl = pl.ds(q * 16, 16)
            rows[b, i * 4 + r, sl] = jnp.maximum(
                rows[b, i * 4 + r, sl] + emb[b, i * 4 + r, sl], 0.0)
        return 0
      lax.fori_loop(0, CH // 4, cbody, 0)
      pltpu.sync_copy(rows.at[b], acc_s.at[dstv.at[b, 0]], add=True)

    start_idx(0, 0)
    start_idx(1, 1)
    wait_idx(0)
    start_data(0, 0)

    def body(jj2, _):
      for b in range(2):
        j = jj2 * 2 + b
        wait_data(b)
        wait_idx(1 - b)
        start_data(j + 1, 1 - b)
        compute_scatter(b)
        start_idx(j + 2, b)
      return 0
    lax.fori_loop(0, (cpt - 2) // 2, body, 0)

    wait_data(0)
    wait_idx(1)
    start_data(cpt - 1, 1)
    compute_scatter(0)
    wait_data(1)
    compute_scatter(1)

    plsc.subcore_barrier()

    pltpu.sync_copy(acc_s.at[pl.ds(s * RPT, RPT)],
                    out_h.at[c, pl.ds(s * RPT, RPT)])

  return k(src2d, dst2d, xp, embs)


def _edge_embed(edge_attr, We, be):
  blk = 1600
  grid = (E // blk,)

  def body(ea_ref, we_ref, be_ref, out_ref):
    out_ref[...] = (
        jnp.dot(ea_ref[...], we_ref[...], preferred_element_type=jnp.float32)
        + be_ref[...])

  return pl.pallas_call(
      body,
      grid=grid,
      in_specs=[
          pl.BlockSpec((blk, DE), lambda i: (i, 0)),
          pl.BlockSpec((DE, D), lambda i: (0, 0)),
          pl.BlockSpec((1, D), lambda i: (0, 0)),
      ],
      out_specs=pl.BlockSpec((blk, D), lambda i: (i, 0)),
      out_shape=jax.ShapeDtypeStruct((EP, D), jnp.float32),
  )(edge_attr, We, be.reshape(1, D))


def _layernorm(a, g, b):
  mu = jnp.mean(a, axis=-1, keepdims=True)
  var = jnp.mean(jnp.square(a - mu), axis=-1, keepdims=True)
  return (a - mu) * lax.rsqrt(var + 1e-5) * g + b


def _mlp(partials, xp, W1, b1, g, bt, W2, b2):
  blk = 2000
  grid = (N // blk,)

  def body(p_ref, x_ref, w1, b1r, gr, btr, w2, b2r, out_ref):
    h = (1.0 + EPS) * x_ref[...] + p_ref[0] + p_ref[1]
    a = jnp.dot(h, w1[...], preferred_element_type=jnp.float32) + b1r[...]
    a = _layernorm(a, gr[...], btr[...])
    a = jnp.maximum(a, 0.0)
    out_ref[...] = (
        jnp.dot(a, w2[...], preferred_element_type=jnp.float32) + b2r[...])

  return pl.pallas_call(
      body,
      grid=grid,
      in_specs=[
          pl.BlockSpec((NC, blk, D), lambda i: (0, i, 0)),
          pl.BlockSpec((blk, D), lambda i: (i, 0)),
          pl.BlockSpec((D, H), lambda i: (0, 0)),
          pl.BlockSpec((1, H), lambda i: (0, 0)),
          pl.BlockSpec((1, H), lambda i: (0, 0)),
          pl.BlockSpec((1, H), lambda i: (0, 0)),
          pl.BlockSpec((H, D), lambda i: (0, 0)),
          pl.BlockSpec((1, D), lambda i: (0, 0)),
      ],
      out_specs=pl.BlockSpec((blk, D), lambda i: (i, 0)),
      out_shape=jax.ShapeDtypeStruct((N, D), jnp.float32),
  )(partials, xp, W1, b1.reshape(1, H), g.reshape(1, H), bt.reshape(1, H),
    W2, b2.reshape(1, D))


def _mlp_final(partials, y1, W1, b1, g, bt, W2, b2, Wout, bout):
  blk = 2000
  grid = (N // blk,)

  def body(p_ref, y1_ref, w1, b1r, gr, btr, w2, b2r, wo, bo, out_ref):
    y1b = y1_ref[...]
    h = (1.0 + EPS) * y1b + p_ref[0] + p_ref[1]
    a = jnp.dot(h, w1[...], preferred_element_type=jnp.float32) + b1r[...]
    a = _layernorm(a, gr[...], btr[...])
    a = jnp.maximum(a, 0.0)
    y2 = jnp.dot(a, w2[...], preferred_element_type=jnp.float32) + b2r[...]
    out_ref[...] = (
        jnp.dot(y1b, wo[:D], preferred_element_type=jnp.float32)
        + jnp.dot(y2, wo[D:], preferred_element_type=jnp.float32)
        + bo[...])

  return pl.pallas_call(
      body,
      grid=grid,
      in_specs=[
          pl.BlockSpec((NC, blk, D), lambda i: (0, i, 0)),
          pl.BlockSpec((blk, D), lambda i: (i, 0)),
          pl.BlockSpec((D, H), lambda i: (0, 0)),
          pl.BlockSpec((1, H), lambda i: (0, 0)),
          pl.BlockSpec((1, H), lambda i: (0, 0)),
          pl.BlockSpec((1, H), lambda i: (0, 0)),
          pl.BlockSpec((H, D), lambda i: (0, 0)),
          pl.BlockSpec((1, D), lambda i: (0, 0)),
          pl.BlockSpec((2 * D, D), lambda i: (0, 0)),
          pl.BlockSpec((1, D), lambda i: (0, 0)),
      ],
      out_specs=pl.BlockSpec((blk, D), lambda i: (i, 0)),
      out_shape=jax.ShapeDtypeStruct((N, D), jnp.float32),
  )(partials, y1, W1, b1.reshape(1, H), g.reshape(1, H), bt.reshape(1, H),
    W2, b2.reshape(1, D), Wout, bout.reshape(1, D))


def kernel(x, edge_index, edge_attr,
           We0, be0, W10, b10, g0, bt0, W20, b20,
           We1, be1, W11, b11, g1, bt1, W21, b21,
           Wout, bout):
  src = edge_index[0]
  dst = edge_index[1]
  pad = EP - E
  src_p = jnp.concatenate([src, jnp.zeros((pad,), jnp.int32)])
  sink = N + jnp.arange(pad, dtype=jnp.int32) % (NSP - N)
  dst_p = jnp.concatenate([dst, sink])
  src2d = src_p.reshape(EP // CH, CH)
  dst2d = dst_p.reshape(EP // CH, CH)
  embs0 = _edge_embed(edge_attr, We0, be0)
  embs1 = _edge_embed(edge_attr, We1, be1)

  part0 = _sc_aggregate(src2d, dst2d, x, embs0)
  y1 = _mlp(part0, x, W10, b10, g0, bt0, W20, b20)
  part1 = _sc_aggregate(src2d, dst2d, y1, embs1)
  return _mlp_final(part1, y1, W11, b11, g1, bt1, W21, b21, Wout, bout)


# speedup vs baseline: 1.1417x; 1.0269x over previous
import functools
import jax
import jax.numpy as jnp
from jax import lax
from jax.experimental import pallas as pl
from jax.experimental.pallas import tpu as pltpu
from jax.experimental.pallas import tpu_sc as plsc

N = 10000
E = 320000
D = 128
DE = 16
H = 256
EPS = 128.0

NC = 2
NT = 16
CH = 64
CPT0 = 228
CPT1 = 92
EP = CH * NT * (CPT0 + CPT1)
NSP = 10240
RPT = NSP // NT


def _sc_aggregate(src2d, dst2d, xp, embs):
  mesh = plsc.VectorSubcoreMesh(core_axis_name="c", subcore_axis_name="s")

  @functools.partial(
      pl.kernel,
      out_type=jax.ShapeDtypeStruct((NC, NSP, D), jnp.float32),
      mesh=mesh,
      scratch_types=[
          pltpu.VMEM_SHARED((NSP, D), jnp.float32),
          pltpu.VMEM((2, 1, CH), jnp.int32),
          pltpu.VMEM((2, 1, CH), jnp.int32),
          pltpu.VMEM((2, CH, D), jnp.float32),
          pltpu.VMEM((2, CH, D), jnp.float32),
          pltpu.SemaphoreType.DMA,
          pltpu.SemaphoreType.DMA,
          pltpu.SemaphoreType.DMA,
          pltpu.SemaphoreType.DMA,
      ],
  )
  def k(src_h, dst_h, xp_h, embs_h, out_h, acc_s, srcv, dstv, rows, emb,
        sem_i0, sem_i1, sem_d0, sem_d1):
    c = lax.axis_index("c")
    s = lax.axis_index("s")
    sem_i = (sem_i0, sem_i1)
    sem_d = (sem_d0, sem_d1)

    def zbody(i, _):
      for q in range(D // 16):
        rows[0, i, pl.ds(q * 16, 16)] = jnp.zeros((16,), jnp.float32)
      return 0
    lax.fori_loop(0, CH, zbody, 0)
    arow = s * RPT
    for t in range(RPT // CH):
      pltpu.sync_copy(rows.at[0, pl.ds(0, CH)],
                      acc_s.at[pl.ds(arow + t * CH, CH)])

    plsc.subcore_barrier()

    cpt = jnp.where(c == 0, CPT0, CPT1)
    tile0 = jnp.where(c == 0, s * CPT0, NT * CPT0 + s * CPT1)

    def start_idx(j, b):
      pltpu.async_copy(src_h.at[pl.ds(tile0 + j, 1)], srcv.at[b], sem_i[b])
      pltpu.async_copy(dst_h.at[pl.ds(tile0 + j, 1)], dstv.at[b], sem_i[b])

    def wait_idx(b):
      pltpu.make_async_copy(src_h.at[pl.ds(0, 1)], srcv.at[b],
                            sem_i[b]).wait()
      pltpu.make_async_copy(dst_h.at[pl.ds(0, 1)], dstv.at[b],
                            sem_i[b]).wait()

    def start_data(j, b):
      pltpu.async_copy(xp_h.at[srcv.at[b, 0]], rows.at[b], sem_d[b])
      pltpu.async_copy(embs_h.at[pl.ds((tile0 + j) * CH, CH)], emb.at[b],
                       sem_d[b])

    def wait_data(b):
      pltpu.make_async_copy(xp_h.at[pl.ds(0, CH)], rows.at[b],
                            sem_d[b]).wait()
      pltpu.make_async_copy(embs_h.at[pl.ds(0, CH)], emb.at[b],
                            sem_d[b]).wait()

    def compute_scatter(b):
      def cbody(i, _):
        for r in range(4):
          for q in range(D // 16):
            sl = pl.ds(q * 16, 16)
            rows[b, i * 4 + r, sl] = jnp.maximum(
                rows[b, i * 4 + r, sl] + emb[b, i * 4 + r, sl], 0.0)
        return 0
      lax.fori_loop(0, CH // 4, cbody, 0)
      pltpu.sync_copy(rows.at[b], acc_s.at[dstv.at[b, 0]], add=True)

    start_idx(0, 0)
    start_idx(1, 1)
    wait_idx(0)
    start_data(0, 0)

    def body(jj2, _):
      for b in range(2):
        j = jj2 * 2 + b
        wait_data(b)
        wait_idx(1 - b)
        start_data(j + 1, 1 - b)
        compute_scatter(b)
        start_idx(j + 2, b)
      return 0
    lax.fori_loop(0, (cpt - 2) // 2, body, 0)

    wait_data(0)
    wait_idx(1)
    start_data(cpt - 1, 1)
    compute_scatter(0)
    wait_data(1)
    compute_scatter(1)

    plsc.subcore_barrier()

    pltpu.sync_copy(acc_s.at[pl.ds(s * RPT, RPT)],
                    out_h.at[c, pl.ds(s * RPT, RPT)])

  return k(src2d, dst2d, xp, embs)


def _edge_embed(edge_attr, We, be):
  blk = 1600
  grid = (E // blk,)

  def body(ea_ref, we_ref, be_ref, out_ref):
    out_ref[...] = (
        jnp.dot(ea_ref[...], we_ref[...], preferred_element_type=jnp.float32)
        + be_ref[...])

  return pl.pallas_call(
      body,
      grid=grid,
      in_specs=[
          pl.BlockSpec((blk, DE), lambda i: (i, 0)),
          pl.BlockSpec((DE, D), lambda i: (0, 0)),
          pl.BlockSpec((1, D), lambda i: (0, 0)),
      ],
      out_specs=pl.BlockSpec((blk, D), lambda i: (i, 0)),
      out_shape=jax.ShapeDtypeStruct((EP, D), jnp.float32),
  )(edge_attr, We, be.reshape(1, D))


def _layernorm(a, g, b):
  mu = jnp.mean(a, axis=-1, keepdims=True)
  var = jnp.mean(jnp.square(a - mu), axis=-1, keepdims=True)
  return (a - mu) * lax.rsqrt(var + 1e-5) * g + b


def _mlp(partials, xp, W1, b1, g, bt, W2, b2):
  blk = 2000
  grid = (N // blk,)

  def body(p_ref, x_ref, w1, b1r, gr, btr, w2, b2r, out_ref):
    h = (1.0 + EPS) * x_ref[...] + p_ref[0] + p_ref[1]
    a = jnp.dot(h, w1[...], preferred_element_type=jnp.float32) + b1r[...]
    a = _layernorm(a, gr[...], btr[...])
    a = jnp.maximum(a, 0.0)
    out_ref[...] = (
        jnp.dot(a, w2[...], preferred_element_type=jnp.float32) + b2r[...])

  return pl.pallas_call(
      body,
      grid=grid,
      in_specs=[
          pl.BlockSpec((NC, blk, D), lambda i: (0, i, 0)),
          pl.BlockSpec((blk, D), lambda i: (i, 0)),
          pl.BlockSpec((D, H), lambda i: (0, 0)),
          pl.BlockSpec((1, H), lambda i: (0, 0)),
          pl.BlockSpec((1, H), lambda i: (0, 0)),
          pl.BlockSpec((1, H), lambda i: (0, 0)),
          pl.BlockSpec((H, D), lambda i: (0, 0)),
          pl.BlockSpec((1, D), lambda i: (0, 0)),
      ],
      out_specs=pl.BlockSpec((blk, D), lambda i: (i, 0)),
      out_shape=jax.ShapeDtypeStruct((N, D), jnp.float32),
  )(partials, xp, W1, b1.reshape(1, H), g.reshape(1, H), bt.reshape(1, H),
    W2, b2.reshape(1, D))


def _mlp_final(partials, y1, W1, b1, g, bt, W2, b2, Wout, bout):
  blk = 2000
  grid = (N // blk,)

  def body(p_ref, y1_ref, w1, b1r, gr, btr, w2, b2r, wo, bo, out_ref):
    y1b = y1_ref[...]
    h = (1.0 + EPS) * y1b + p_ref[0] + p_ref[1]
    a = jnp.dot(h, w1[...], preferred_element_type=jnp.float32) + b1r[...]
    a = _layernorm(a, gr[...], btr[...])
    a = jnp.maximum(a, 0.0)
    y2 = jnp.dot(a, w2[...], preferred_element_type=jnp.float32) + b2r[...]
    out_ref[...] = (
        jnp.dot(y1b, wo[:D], preferred_element_type=jnp.float32)
        + jnp.dot(y2, wo[D:], preferred_element_type=jnp.float32)
        + bo[...])

  return pl.pallas_call(
      body,
      grid=grid,
      in_specs=[
          pl.BlockSpec((NC, blk, D), lambda i: (0, i, 0)),
          pl.BlockSpec((blk, D), lambda i: (i, 0)),
          pl.BlockSpec((D, H), lambda i: (0, 0)),
          pl.BlockSpec((1, H), lambda i: (0, 0)),
          pl.BlockSpec((1, H), lambda i: (0, 0)),
          pl.BlockSpec((1, H), lambda i: (0, 0)),
          pl.BlockSpec((H, D), lambda i: (0, 0)),
          pl.BlockSpec((1, D), lambda i: (0, 0)),
          pl.BlockSpec((2 * D, D), lambda i: (0, 0)),
          pl.BlockSpec((1, D), lambda i: (0, 0)),
      ],
      out_specs=pl.BlockSpec((blk, D), lambda i: (i, 0)),
      out_shape=jax.ShapeDtypeStruct((N, D), jnp.float32),
  )(partials, y1, W1, b1.reshape(1, H), g.reshape(1, H), bt.reshape(1, H),
    W2, b2.reshape(1, D), Wout, bout.reshape(1, D))


def kernel(x, edge_index, edge_attr,
           We0, be0, W10, b10, g0, bt0, W20, b20,
           We1, be1, W11, b11, g1, bt1, W21, b21,
           Wout, bout):
  src = edge_index[0]
  dst = edge_index[1]
  pad = EP - E
  src_p = jnp.concatenate([src, jnp.zeros((pad,), jnp.int32)])
  sink = N + jnp.arange(pad, dtype=jnp.int32) % (NSP - N)
  dst_p = jnp.concatenate([dst, sink])
  src2d = src_p.reshape(EP // CH, CH)
  dst2d = dst_p.reshape(EP // CH, CH)
  embs0 = _edge_embed(edge_attr, We0, be0)
  embs1 = _edge_embed(edge_attr, We1, be1)

  part0 = _sc_aggregate(src2d, dst2d, x, embs0)
  y1 = _mlp(part0, x, W10, b10, g0, bt0, W20, b20)
  part1 = _sc_aggregate(src2d, dst2d, y1, embs1)
  return _mlp_final(part1, y1, W11, b11, g1, bt1, W21, b21, Wout, bout)
